# baseline (speedup 1.0000x reference)
import functools

import jax
import jax.numpy as jnp
from jax import lax
from jax.experimental import pallas as pl
from jax.experimental.pallas import tpu as pltpu
from jax.experimental.pallas import tpu_sc as plsc

NUM_EMB = 1_000_000
D = 32
BATCH = 16384
N_FIELDS = 26
B_TOTAL = BATCH * N_FIELDS

NC = 2
NS = 16
NW = NC * NS
B_PER_W = B_TOTAL // NW
G = 128
NG = B_PER_W // G
K = 8
NS = 16


def _build():
  mesh = plsc.VectorSubcoreMesh(core_axis_name="c", subcore_axis_name="s")

  @functools.partial(
      pl.kernel,
      mesh=mesh,
      out_type=jax.ShapeDtypeStruct((B_TOTAL, D), jnp.float32),
      scratch_types=[
          pltpu.VMEM((NG, G), jnp.int32),
          pltpu.VMEM((NS, G, D), jnp.float32),
          pltpu.SemaphoreType.DMA,
          pltpu.SemaphoreType.DMA,
      ],
      compiler_params=pltpu.CompilerParams(use_tc_tiling_on_sc=False),
  )
  def emb_kernel(table_hbm, idx_hbm, out_hbm, idx_v, rows_v, gsem, wsem):
    wid = lax.axis_index("s") * NC + lax.axis_index("c")
    base = wid * B_PER_W
    pltpu.sync_copy(idx_hbm.at[wid], idx_v)

    for b in range(K):
      pltpu.async_copy(table_hbm.at[idx_v.at[b]], rows_v.at[b], gsem)

    def outer(o, carry):
      for b in range(K):
        i = o * K + b
        s = lax.rem(i, NS)
        pltpu.make_async_copy(
            table_hbm.at[pl.ds(0, G)], rows_v.at[s], gsem
        ).wait()
        pltpu.async_copy(rows_v.at[s], out_hbm.at[pl.ds(base + i * G, G)], wsem)

        @pl.when(i >= K)
        def _():
          pltpu.make_async_copy(
              rows_v.at[0], out_hbm.at[pl.ds(base, G)], wsem
          ).wait()

        nxt = i + K

        @pl.when(nxt < NG)
        def _():
          pltpu.async_copy(
              table_hbm.at[idx_v.at[nxt]], rows_v.at[lax.rem(nxt, NS)], gsem
          )

      return carry

    lax.fori_loop(0, NG // K, outer, 0)

    for _ in range(K):
      pltpu.make_async_copy(rows_v.at[0], out_hbm.at[pl.ds(base, G)], wsem).wait()

  return emb_kernel


def kernel(indices, table):
  idx = indices.reshape(NW, NG, G).astype(jnp.int32)
  out = _build()(table, idx)
  return out.reshape(BATCH, N_FIELDS, D)


# speedup vs baseline: 1.0619x; 1.0619x over previous
import functools

import jax
import jax.numpy as jnp
from jax import lax
from jax.experimental import pallas as pl
from jax.experimental.pallas import tpu as pltpu
from jax.experimental.pallas import tpu_sc as plsc

NUM_EMB = 1_000_000
D = 32
BATCH = 16384
N_FIELDS = 26
B_TOTAL = BATCH * N_FIELDS

NC = 2
NT = 16
NW = NC * NT
B_PER_W = B_TOTAL // NW
G = 128
NG = B_PER_W // G
K = 8
NS = 16


def _build():
  mesh = plsc.VectorSubcoreMesh(core_axis_name="c", subcore_axis_name="s")

  @functools.partial(
      pl.kernel,
      mesh=mesh,
      out_type=jax.ShapeDtypeStruct((B_TOTAL, D), jnp.float32),
      scratch_types=[
          pltpu.VMEM((NG, G), jnp.int32),
          pltpu.VMEM((NS, G, D), jnp.float32),
          pltpu.SemaphoreType.DMA,
          pltpu.SemaphoreType.DMA,
      ],
      compiler_params=pltpu.CompilerParams(use_tc_tiling_on_sc=False),
  )
  def emb_kernel(table_hbm, idx_hbm, out_hbm, idx_v, rows_v, gsem, wsem):
    wid = lax.axis_index("s") * NC + lax.axis_index("c")
    base = wid * B_PER_W
    pltpu.sync_copy(idx_hbm.at[wid], idx_v)

    for b in range(K):
      pltpu.async_copy(table_hbm.at[idx_v.at[b]], rows_v.at[b], gsem)

    def outer(o, carry):
      for b in range(K):
        i = o * K + b
        s = lax.rem(i, NS)
        pltpu.make_async_copy(
            table_hbm.at[pl.ds(0, G)], rows_v.at[s], gsem
        ).wait()
        pltpu.async_copy(rows_v.at[s], out_hbm.at[pl.ds(base + i * G, G)], wsem)

        @pl.when(i >= K)
        def _():
          pltpu.make_async_copy(
              rows_v.at[0], out_hbm.at[pl.ds(base, G)], wsem
          ).wait()

        nxt = i + K

        @pl.when(nxt < NG)
        def _():
          pltpu.async_copy(
              table_hbm.at[idx_v.at[nxt]], rows_v.at[lax.rem(nxt, NS)], gsem
          )

      return carry

    lax.fori_loop(0, NG // K, outer, 0)

    for _ in range(K):
      pltpu.make_async_copy(rows_v.at[0], out_hbm.at[pl.ds(base, G)], wsem).wait()

  return emb_kernel


def kernel(indices, table):
  idx = indices.T.astype(jnp.int32).reshape(NW, NG, G)
  out = _build()(table, idx)
  return out.reshape(N_FIELDS, BATCH, D).transpose(1, 0, 2)
